# Initial kernel scaffold; baseline (speedup 1.0000x reference)
import jax
import jax.numpy as jnp
from jax.experimental import pallas as pl


def kernel(gene_expression, pert_idx, graph_batch_indices, G_coexpress, G_coexpress_weight, G_go, G_go_weight, params):
    raise NotImplementedError("write your pallas kernel here")


# speedup vs baseline: 27.4934x; 27.4934x over previous
import functools
import jax
import jax.numpy as jnp
from jax import lax
from jax.experimental import pallas as pl
from jax.experimental.pallas import tpu as pltpu
from jax.experimental.pallas import tpu_sc as plsc

B = 32
G = 10000
P = 5000
H = 64
ECO = 320000
EGO = 160000
NC = 2
NS = 16
NW = NC * NS
CH = 128

EWC = 10112
EWG = 5120
NCHC = EWC // CH
NCHG = EWG // CH
GN = 10112
PN = 5120


def _make_sc_deg(npad, eww, nch):
    rp = npad // NS
    mesh = plsc.VectorSubcoreMesh(core_axis_name="c", subcore_axis_name="s")

    @functools.partial(
        pl.kernel, mesh=mesh,
        compiler_params=pltpu.CompilerParams(needs_layout_passes=False,
                                             use_tc_tiling_on_sc=False),
        out_type=jax.ShapeDtypeStruct((NC, npad, 16), jnp.float32),
        scratch_types=[
            pltpu.VMEM((nch, CH), jnp.int32),
            pltpu.VMEM((eww,), jnp.float32),
            pltpu.VMEM((CH, 16), jnp.float32),
            pltpu.VMEM_SHARED((npad, 16), jnp.float32),
        ])
    def k(dst3, ew2, zer, out, dstv, ewv, upd, acc):
        cid = lax.axis_index("c")
        sid = lax.axis_index("s")
        wid = sid * NC + cid
        pltpu.sync_copy(dst3.at[wid], dstv)
        pltpu.sync_copy(ew2.at[wid], ewv)
        pltpu.sync_copy(zer.at[pl.ds(sid * rp, rp)], acc.at[pl.ds(sid * rp, rp)])
        plsc.subcore_barrier()

        def chunk(j, carry):
            def srow(r, c2):
                sp = plsc.load_gather(
                    ewv, [jnp.full((16,), j * CH + r, jnp.int32)])
                upd[r, pl.ds(0, 16)] = sp
                return c2

            lax.fori_loop(0, CH, srow, 0)
            pltpu.sync_copy(upd, acc.at[dstv.at[j]], add=True)
            return carry

        lax.fori_loop(0, nch, chunk, 0)
        plsc.subcore_barrier()
        pltpu.sync_copy(acc.at[pl.ds(sid * rp, rp)],
                        out.at[cid, pl.ds(sid * rp, rp)])

    return k


def _make_sc_agg(npad, eww, nch):
    rp = npad // NS
    mesh = plsc.VectorSubcoreMesh(core_axis_name="c", subcore_axis_name="s")

    @functools.partial(
        pl.kernel, mesh=mesh,
        compiler_params=pltpu.CompilerParams(needs_layout_passes=False,
                                             use_tc_tiling_on_sc=False),
        out_type=jax.ShapeDtypeStruct((NC, npad, H), jnp.float32),
        scratch_types=[
            pltpu.VMEM((nch, CH), jnp.int32),
            pltpu.VMEM((nch, CH), jnp.int32),
            pltpu.VMEM((eww,), jnp.float32),
            pltpu.VMEM((CH, H), jnp.float32),
            pltpu.VMEM_SHARED((npad, H), jnp.float32),
            pltpu.SemaphoreType.DMA,
        ])
    def k(xp, src3, dst3, ew2, zer, out, srcv, dstv, ewv, rows, acc, sem):
        cid = lax.axis_index("c")
        sid = lax.axis_index("s")
        wid = sid * NC + cid
        pltpu.sync_copy(src3.at[wid], srcv)
        pltpu.sync_copy(dst3.at[wid], dstv)
        pltpu.sync_copy(ew2.at[wid], ewv)
        pltpu.sync_copy(zer.at[pl.ds(sid * rp, rp)], acc.at[pl.ds(sid * rp, rp)])
        plsc.subcore_barrier()

        def chunk(j, carry):
            pltpu.async_copy(xp.at[srcv.at[j]], rows, sem).wait()

            def srow(r, c2):
                sp = plsc.load_gather(
                    ewv, [jnp.full((16,), j * CH + r, jnp.int32)])
                for q in range(H // 16):
                    rows[r, pl.ds(q * 16, 16)] = rows[r, pl.ds(q * 16, 16)] * sp
                return c2

            lax.fori_loop(0, CH, srow, 0)
            pltpu.sync_copy(rows, acc.at[dstv.at[j]], add=True)
            return carry

        lax.fori_loop(0, nch, chunk, 0)
        plsc.subcore_barrier()
        pltpu.sync_copy(acc.at[pl.ds(sid * rp, rp)],
                        out.at[cid, pl.ds(sid * rp, rp)])

    return k


_sc_deg_co = _make_sc_deg(GN, EWC, NCHC)
_sc_deg_go = _make_sc_deg(PN, EWG, NCHG)
_sc_agg_co = _make_sc_agg(GN, EWC, NCHC)
_sc_agg_go = _make_sc_agg(PN, EWG, NCHG)


def _normrows(x):
    n = jnp.sqrt(jnp.sum(x * x, axis=1, keepdims=True))
    return x * jnp.where(n > 1.0, 1.0 / jnp.maximum(n, 1e-7), 1.0)


def _dot(a, b):
    return jnp.dot(a, b, preferred_element_type=jnp.float32)


def _tc_prep_body(ge, cop, gop, bng, bnb,
                  ab_o, xpco_o, xd_o, xpgo_o, xdg_o):
    xn = _normrows(ge[...])
    m = jnp.mean(xn, axis=0, keepdims=True)
    v = jnp.mean(xn * xn, axis=0, keepdims=True) - m * m
    base = jnp.maximum(
        (xn - m) / jnp.sqrt(v + 1e-5) * bng[...] + bnb[...], 0.0)

    xp = _normrows(cop[:, :H])
    deg = 1.0 + cop[:, H:H + 1] + cop[:, H + 1:H + 2]
    dinv = lax.rsqrt(deg)
    ab_o[...] = jnp.concatenate([base, xp], axis=1)
    xpco_o[...] = xp * dinv
    xd_o[...] = jnp.concatenate(
        [xp * (1.0 / deg), dinv, jnp.zeros((G, H - 1), jnp.float32)], axis=1)

    xq = _normrows(gop[:, :H])
    degp = 1.0 + gop[:, H:H + 1] + gop[:, H + 1:H + 2]
    dinvp = lax.rsqrt(degp)
    xpgo_o[...] = xq * dinvp
    xdg_o[...] = jnp.concatenate(
        [xq * (1.0 / degp), dinvp, jnp.zeros((P, H - 1), jnp.float32)], axis=1)


def _tc_mida_body(spc, ab, xd,
                  wco, bco,
                  e2w1, e2b1, e2g, e2b, e2w2, e2b2,
                  up_o):
    base = ab[:, :H]
    xpos = ab[:, H:]
    agg = xd[:, H:H + 1] * (spc[:, :H] + spc[:, H:]) + xd[:, :H]
    pos0 = _dot(agg, wco[...]) + bco[...]
    pos1 = _dot(xpos, wco[...]) + bco[...]
    t0 = base + 0.2 * pos0
    t1 = base + 0.2 * pos1

    h0 = _dot(t0, e2w1[...]) + e2b1[...]
    h1 = _dot(t1, e2w1[...]) + e2b1[...]
    nbg = float(B * G)
    s1 = (jnp.sum(h0, 0, keepdims=True)
          + (B - 1) * jnp.sum(h1, 0, keepdims=True)) / nbg
    s2 = (jnp.sum(h0 * h0, 0, keepdims=True)
          + (B - 1) * jnp.sum(h1 * h1, 0, keepdims=True)) / nbg
    sc = e2g[...] / jnp.sqrt(s2 - s1 * s1 + 1e-5)
    a0 = jnp.maximum((h0 - s1) * sc + e2b[...], 0.0)
    a1 = jnp.maximum((h1 - s1) * sc + e2b[...], 0.0)
    u0 = jnp.maximum(_dot(a0, e2w2[...]) + e2b2[...], 0.0)
    u1 = jnp.maximum(_dot(a1, e2w2[...]) + e2b2[...], 0.0)
    up_o[...] = jnp.concatenate([u0, u1], axis=1)


def _tc_midb_body(up, spg, xdg, pidx,
                  wgo, bgo,
                  pfw1, pfb1, pfg, pfb, pfw2, pfb2,
                  pbg, pbb,
                  vp_o, f_o):
    pg = _dot(xdg[:, H:H + 1] * (spg[:, :H] + spg[:, H:]) + xdg[:, :H],
              wgo[...]) + bgo[...]
    iot = lax.broadcasted_iota(jnp.int32, (2 * B, P), 1)
    oh = (iot == pidx[...]).astype(jnp.float32)
    rows = _dot(oh, pg)
    ps = rows.reshape(B, 2, H).sum(axis=1)
    hp = _dot(ps, pfw1[...]) + pfb1[...]
    mp = jnp.mean(hp, 0, keepdims=True)
    vp = jnp.mean(hp * hp, 0, keepdims=True) - mp * mp
    hp = (hp - mp) / jnp.sqrt(vp + 1e-5) * pfg[...] + pfb[...]
    e = jnp.maximum(_dot(jnp.maximum(hp, 0.0), pfw2[...]) + pfb2[...], 0.0)

    u0 = up[:, :H]
    u1 = up[:, H:]
    nbg = float(B * G)
    su0 = jnp.sum(u0, 0, keepdims=True)
    su1 = jnp.sum(u1, 0, keepdims=True)
    sq0 = jnp.sum(u0 * u0, 0, keepdims=True)
    sq1 = jnp.sum(u1 * u1, 0, keepdims=True)
    se = jnp.sum(e, 0, keepdims=True)
    e0 = e[0:1]
    se1 = se - e0
    see = jnp.sum(e * e, 0, keepdims=True) - e0 * e0
    m = (su0 + (B - 1) * su1) / nbg + se / B
    ex2 = (sq0 + 2.0 * e0 * su0 + G * e0 * e0
           + (B - 1) * sq1 + 2.0 * se1 * su1 + G * see) / nbg
    scpb = pbg[...] / jnp.sqrt(ex2 - m * m + 1e-5)
    off = pbb[...] - m * scpb
    vp_o[...] = jnp.concatenate([u0 * scpb + off, u1 * scpb + off], axis=1)
    f_o[...] = e * scpb


def _tc_pass1_body(vp, f, sum_o, m2_o):
    b = pl.program_id(0)
    v = jnp.where(b == 0, vp[:, :H], vp[:, H:])
    r = jnp.maximum(v + f[0], 0.0)

    @pl.when(b == 0)
    def _():
        sum_o[...] = jnp.zeros_like(sum_o)
        m2_o[...] = jnp.zeros_like(m2_o)

    sum_o[...] += jnp.sum(r, 0, keepdims=True)
    m2_o[...] += lax.dot_general(r, r, (((0,), (0,)), ((), ())),
                                 preferred_element_type=jnp.float32)


def _tc_pass2_body(vp, f, sum_r, m2, w1, b1, g1, bb1, w2, b2,
                   iw1, o1_o):
    nbg = float(B * G)
    mean_r = sum_r[...] / nbg
    a = m2[...] / nbg
    mr_w = _dot(mean_r, w1[...])
    mh = mr_w + b1[...]
    t = _dot(a, w1[...])
    ex2 = (jnp.sum(w1[...] * t, 0, keepdims=True)
           + 2.0 * b1[...] * mr_w + b1[...] * b1[...])
    scr = g1[...] / jnp.sqrt(ex2 - mh * mh + 1e-5)

    b = pl.program_id(0)
    v = jnp.where(b == 0, vp[:, :H], vp[:, H:])
    r = jnp.maximum(v + f[0], 0.0)
    h = _dot(r, w1[...]) + b1[...]
    h = jnp.maximum((h - mh) * scr + bb1[...], 0.0)
    oh = _dot(h, w2[...]) + b2[...]
    o1_o[...] = jnp.sum(oh * iw1[...], 1, keepdims=True)


def _tc_final_body(o1, ib1r, cgw1, cgg, cgb, cgw2, cgb2, w2a, w2bt, ib2,
                   xres, out_o):
    o1b = o1[...] + ib1r[...]
    cgp = _dot(o1b, cgw1[...])
    m = jnp.mean(cgp, 0, keepdims=True)
    v = jnp.mean(cgp * cgp, 0, keepdims=True) - m * m
    c = jnp.maximum((cgp - m) / jnp.sqrt(v + 1e-5) * cgg[...] + cgb[...], 0.0)
    cgv = _dot(c, cgw2[...]) + cgb2[...]
    out_o[...] = (o1b * w2a[...] + _dot(cgv, w2bt[...])
                  + ib2[...] + xres[...])


def _vspec(shape, imap=None):
    if imap is None:
        return pl.BlockSpec(shape, lambda b: tuple(0 for _ in shape))
    return pl.BlockSpec(shape, imap)


def _pad_edges(src, dst, ew, e_pad, n_nodes, nch):
    npd = e_pad - src.shape[0]
    fill = (jnp.arange(npd, dtype=jnp.int32) % n_nodes).astype(jnp.int32)
    src_p = jnp.concatenate([src.astype(jnp.int32), fill])
    dst_p = jnp.concatenate([dst.astype(jnp.int32), fill])
    ew_p = jnp.concatenate([ew, jnp.zeros((npd,), jnp.float32)])
    eww = e_pad // NW
    return (src_p.reshape(NW, nch, CH), dst_p.reshape(NW, nch, CH),
            ew_p.reshape(NW, eww))


def kernel(gene_expression, pert_idx, graph_batch_indices, G_coexpress,
           G_coexpress_weight, G_go, G_go_weight, params):
    del graph_batch_indices
    p = params
    f32 = jnp.float32

    src3c, dst3c, ew2c = _pad_edges(G_coexpress[0], G_coexpress[1],
                                    G_coexpress_weight, EWC * NW, G, NCHC)
    src3g, dst3g, ew2g = _pad_edges(G_go[0], G_go[1],
                                    G_go_weight, EWG * NW, P, NCHG)

    zer16c = jnp.zeros((GN, 16), f32)
    zer16g = jnp.zeros((PN, 16), f32)
    zer64c = jnp.zeros((GN, H), f32)
    zer64g = jnp.zeros((PN, H), f32)

    degc = _sc_deg_co(dst3c, ew2c, zer16c)
    degg = _sc_deg_go(dst3g, ew2g, zer16g)

    r1 = lambda a: a.reshape(1, -1)
    co_pack = jnp.concatenate(
        [p['emb_pos'], degc[0, :G, 0:1], degc[1, :G, 0:1]], axis=1)
    go_pack = jnp.concatenate(
        [p['pert_emb'], degg[0, :P, 0:1], degg[1, :P, 0:1]], axis=1)

    prep = pl.pallas_call(
        _tc_prep_body,
        out_shape=[jax.ShapeDtypeStruct((G, 2 * H), f32),
                   jax.ShapeDtypeStruct((G, H), f32),
                   jax.ShapeDtypeStruct((G, 2 * H), f32),
                   jax.ShapeDtypeStruct((P, H), f32),
                   jax.ShapeDtypeStruct((P, 2 * H), f32)],
    )
    ab, xpco, xd, xpgo, xdg = prep(
        p['gene_emb'], co_pack, go_pack,
        r1(p['bn_emb_g']), r1(p['bn_emb_b']))

    sco = _sc_agg_co(xpco, src3c, dst3c, ew2c, zer64c)
    sgo = _sc_agg_go(xpgo, src3g, dst3g, ew2g, zer64g)
    spc = jnp.concatenate([sco[0, :G], sco[1, :G]], axis=1)
    spg = jnp.concatenate([sgo[0, :P], sgo[1, :P]], axis=1)

    mida = pl.pallas_call(
        _tc_mida_body,
        out_shape=jax.ShapeDtypeStruct((G, 2 * H), f32),
    )
    up = mida(
        spc, ab, xd,
        p['sg_co_W'], r1(p['sg_co_b']),
        p['etv2_W1'], r1(p['etv2_b1']), r1(p['etv2_bng']), r1(p['etv2_bnb']),
        p['etv2_W2'], r1(p['etv2_b2']))

    midb = pl.pallas_call(
        _tc_midb_body,
        out_shape=[jax.ShapeDtypeStruct((G, 2 * H), f32),
                   jax.ShapeDtypeStruct((B, H), f32)],
    )
    vpk, f = midb(
        up, spg, xdg,
        pert_idx.reshape(2 * B, 1).astype(jnp.int32),
        p['sg_go_W'], r1(p['sg_go_b']),
        p['pf_W1'], r1(p['pf_b1']), r1(p['pf_bng']), r1(p['pf_bnb']),
        p['pf_W2'], r1(p['pf_b2']),
        r1(p['bn_pb_g']), r1(p['bn_pb_b']))

    f3 = f.reshape(B, 1, H)
    sum_r, m2 = pl.pallas_call(
        _tc_pass1_body,
        grid=(B,),
        in_specs=[_vspec((G, 2 * H)),
                  _vspec((1, 1, H), lambda b: (b, 0, 0))],
        out_specs=[_vspec((1, H)), _vspec((H, H))],
        out_shape=[jax.ShapeDtypeStruct((1, H), f32),
                   jax.ShapeDtypeStruct((H, H), f32)],
    )(vpk, f3)

    o1c = pl.pallas_call(
        _tc_pass2_body,
        grid=(B,),
        in_specs=[_vspec((G, 2 * H)),
                  _vspec((1, 1, H), lambda b: (b, 0, 0)),
                  _vspec((1, H)), _vspec((H, H)),
                  _vspec((H, 2 * H)), _vspec((1, 2 * H)),
                  _vspec((1, 2 * H)), _vspec((1, 2 * H)),
                  _vspec((2 * H, H)), _vspec((1, H)),
                  _vspec((G, H))],
        out_specs=pl.BlockSpec((G, 1), lambda b: (b, 0)),
        out_shape=jax.ShapeDtypeStruct((B * G, 1), f32),
    )(vpk, f3, sum_r, m2,
      p['rw_W1'], r1(p['rw_b1']), r1(p['rw_bng']), r1(p['rw_bnb']),
      p['rw_W2'], r1(p['rw_b2']),
      p['indv_w1'][:, :, 0])

    final = pl.pallas_call(
        _tc_final_body,
        out_shape=jax.ShapeDtypeStruct((B, G), f32),
    )
    w2 = p['indv_w2'][0]
    return final(o1c.reshape(B, G), p['indv_b1'].reshape(1, G), p['cg_W1'],
                 r1(p['cg_bng']), r1(p['cg_bnb']), p['cg_W2'], r1(p['cg_b2']),
                 w2[:, 0].reshape(1, G), w2[:, 1:].T, p['indv_b2'],
                 gene_expression.reshape(B, G))


# speedup vs baseline: 27.8304x; 1.0123x over previous
import functools
import jax
import jax.numpy as jnp
from jax import lax
from jax.experimental import pallas as pl
from jax.experimental.pallas import tpu as pltpu
from jax.experimental.pallas import tpu_sc as plsc

B = 32
G = 10000
P = 5000
H = 64
ECO = 320000
EGO = 160000
NC = 2
NS = 16
NW = NC * NS
CH = 128

EWC = 10112
EWG = 5120
NCHC = EWC // CH
NCHG = EWG // CH
GN = 10112
PN = 5120


def _make_sc_deg(npad, eww, nch):
    rp = npad // NS
    mesh = plsc.VectorSubcoreMesh(core_axis_name="c", subcore_axis_name="s")

    @functools.partial(
        pl.kernel, mesh=mesh,
        compiler_params=pltpu.CompilerParams(needs_layout_passes=False,
                                             use_tc_tiling_on_sc=False),
        out_type=jax.ShapeDtypeStruct((NC, npad, 16), jnp.float32),
        scratch_types=[
            pltpu.VMEM((nch, CH), jnp.int32),
            pltpu.VMEM((eww,), jnp.float32),
            pltpu.VMEM((CH, 16), jnp.float32),
            pltpu.VMEM_SHARED((npad, 16), jnp.float32),
        ])
    def k(dst3, ew2, zer, out, dstv, ewv, upd, acc):
        cid = lax.axis_index("c")
        sid = lax.axis_index("s")
        wid = sid * NC + cid
        pltpu.sync_copy(dst3.at[wid], dstv)
        pltpu.sync_copy(ew2.at[wid], ewv)
        pltpu.sync_copy(zer.at[pl.ds(sid * rp, rp)], acc.at[pl.ds(sid * rp, rp)])
        plsc.subcore_barrier()

        def chunk(j, carry):
            def srow(r8, c2):
                for u in range(8):
                    r = r8 * 8 + u
                    sp = plsc.load_gather(
                        ewv, [jnp.full((16,), j * CH + r, jnp.int32)])
                    upd[r, pl.ds(0, 16)] = sp
                return c2

            lax.fori_loop(0, CH // 8, srow, 0)
            pltpu.sync_copy(upd, acc.at[dstv.at[j]], add=True)
            return carry

        lax.fori_loop(0, nch, chunk, 0)
        plsc.subcore_barrier()
        pltpu.sync_copy(acc.at[pl.ds(sid * rp, rp)],
                        out.at[cid, pl.ds(sid * rp, rp)])

    return k


def _make_sc_agg(npad, eww, nch):
    rp = npad // NS
    mesh = plsc.VectorSubcoreMesh(core_axis_name="c", subcore_axis_name="s")

    @functools.partial(
        pl.kernel, mesh=mesh,
        compiler_params=pltpu.CompilerParams(needs_layout_passes=False,
                                             use_tc_tiling_on_sc=False),
        out_type=jax.ShapeDtypeStruct((NC, npad, H), jnp.float32),
        scratch_types=[
            pltpu.VMEM((nch, CH), jnp.int32),
            pltpu.VMEM((nch, CH), jnp.int32),
            pltpu.VMEM((eww,), jnp.float32),
            pltpu.VMEM((CH, H), jnp.float32),
            pltpu.VMEM_SHARED((npad, H), jnp.float32),
            pltpu.SemaphoreType.DMA,
        ])
    def k(xp, src3, dst3, ew2, zer, out, srcv, dstv, ewv, rows, acc, sem):
        cid = lax.axis_index("c")
        sid = lax.axis_index("s")
        wid = sid * NC + cid
        pltpu.sync_copy(src3.at[wid], srcv)
        pltpu.sync_copy(dst3.at[wid], dstv)
        pltpu.sync_copy(ew2.at[wid], ewv)
        pltpu.sync_copy(zer.at[pl.ds(sid * rp, rp)], acc.at[pl.ds(sid * rp, rp)])
        plsc.subcore_barrier()

        def chunk(j, carry):
            pltpu.async_copy(xp.at[srcv.at[j]], rows, sem).wait()

            def srow(r4, c2):
                for u in range(4):
                    r = r4 * 4 + u
                    sp = plsc.load_gather(
                        ewv, [jnp.full((16,), j * CH + r, jnp.int32)])
                    for q in range(H // 16):
                        rows[r, pl.ds(q * 16, 16)] = (
                            rows[r, pl.ds(q * 16, 16)] * sp)
                return c2

            lax.fori_loop(0, CH // 4, srow, 0)
            pltpu.sync_copy(rows, acc.at[dstv.at[j]], add=True)
            return carry

        lax.fori_loop(0, nch, chunk, 0)
        plsc.subcore_barrier()
        pltpu.sync_copy(acc.at[pl.ds(sid * rp, rp)],
                        out.at[cid, pl.ds(sid * rp, rp)])

    return k


_sc_deg_co = _make_sc_deg(GN, EWC, NCHC)
_sc_deg_go = _make_sc_deg(PN, EWG, NCHG)
_sc_agg_co = _make_sc_agg(GN, EWC, NCHC)
_sc_agg_go = _make_sc_agg(PN, EWG, NCHG)


def _normrows(x):
    n = jnp.sqrt(jnp.sum(x * x, axis=1, keepdims=True))
    return x * jnp.where(n > 1.0, 1.0 / jnp.maximum(n, 1e-7), 1.0)


def _dot(a, b):
    return jnp.dot(a, b, preferred_element_type=jnp.float32)


def _tc_prep_body(ge, cop, gop, bng, bnb,
                  ab_o, xpco_o, xd_o, xpgo_o, xdg_o):
    xn = _normrows(ge[...])
    m = jnp.mean(xn, axis=0, keepdims=True)
    v = jnp.mean(xn * xn, axis=0, keepdims=True) - m * m
    base = jnp.maximum(
        (xn - m) / jnp.sqrt(v + 1e-5) * bng[...] + bnb[...], 0.0)

    xp = _normrows(cop[:, :H])
    deg = 1.0 + cop[:, H:H + 1] + cop[:, H + 1:H + 2]
    dinv = lax.rsqrt(deg)
    ab_o[...] = jnp.concatenate([base, xp], axis=1)
    xpco_o[...] = xp * dinv
    xd_o[...] = jnp.concatenate(
        [xp * (1.0 / deg), dinv, jnp.zeros((G, H - 1), jnp.float32)], axis=1)

    xq = _normrows(gop[:, :H])
    degp = 1.0 + gop[:, H:H + 1] + gop[:, H + 1:H + 2]
    dinvp = lax.rsqrt(degp)
    xpgo_o[...] = xq * dinvp
    xdg_o[...] = jnp.concatenate(
        [xq * (1.0 / degp), dinvp, jnp.zeros((P, H - 1), jnp.float32)], axis=1)


def _tc_mida_body(spc, ab, xd,
                  wco, bco,
                  e2w1, e2b1, e2g, e2b, e2w2, e2b2,
                  up_o):
    base = ab[:, :H]
    xpos = ab[:, H:]
    agg = xd[:, H:H + 1] * (spc[:, :H] + spc[:, H:]) + xd[:, :H]
    pos0 = _dot(agg, wco[...]) + bco[...]
    pos1 = _dot(xpos, wco[...]) + bco[...]
    t0 = base + 0.2 * pos0
    t1 = base + 0.2 * pos1

    h0 = _dot(t0, e2w1[...]) + e2b1[...]
    h1 = _dot(t1, e2w1[...]) + e2b1[...]
    nbg = float(B * G)
    s1 = (jnp.sum(h0, 0, keepdims=True)
          + (B - 1) * jnp.sum(h1, 0, keepdims=True)) / nbg
    s2 = (jnp.sum(h0 * h0, 0, keepdims=True)
          + (B - 1) * jnp.sum(h1 * h1, 0, keepdims=True)) / nbg
    sc = e2g[...] / jnp.sqrt(s2 - s1 * s1 + 1e-5)
    a0 = jnp.maximum((h0 - s1) * sc + e2b[...], 0.0)
    a1 = jnp.maximum((h1 - s1) * sc + e2b[...], 0.0)
    u0 = jnp.maximum(_dot(a0, e2w2[...]) + e2b2[...], 0.0)
    u1 = jnp.maximum(_dot(a1, e2w2[...]) + e2b2[...], 0.0)
    up_o[...] = jnp.concatenate([u0, u1], axis=1)


def _tc_midb_body(up, spg, xdg, pidx,
                  wgo, bgo,
                  pfw1, pfb1, pfg, pfb, pfw2, pfb2,
                  pbg, pbb,
                  vp_o, f_o):
    pg = _dot(xdg[:, H:H + 1] * (spg[:, :H] + spg[:, H:]) + xdg[:, :H],
              wgo[...]) + bgo[...]
    iot = lax.broadcasted_iota(jnp.int32, (2 * B, P), 1)
    oh = (iot == pidx[...]).astype(jnp.float32)
    rows = _dot(oh, pg)
    ps = rows.reshape(B, 2, H).sum(axis=1)
    hp = _dot(ps, pfw1[...]) + pfb1[...]
    mp = jnp.mean(hp, 0, keepdims=True)
    vp = jnp.mean(hp * hp, 0, keepdims=True) - mp * mp
    hp = (hp - mp) / jnp.sqrt(vp + 1e-5) * pfg[...] + pfb[...]
    e = jnp.maximum(_dot(jnp.maximum(hp, 0.0), pfw2[...]) + pfb2[...], 0.0)

    u0 = up[:, :H]
    u1 = up[:, H:]
    nbg = float(B * G)
    su0 = jnp.sum(u0, 0, keepdims=True)
    su1 = jnp.sum(u1, 0, keepdims=True)
    sq0 = jnp.sum(u0 * u0, 0, keepdims=True)
    sq1 = jnp.sum(u1 * u1, 0, keepdims=True)
    se = jnp.sum(e, 0, keepdims=True)
    e0 = e[0:1]
    se1 = se - e0
    see = jnp.sum(e * e, 0, keepdims=True) - e0 * e0
    m = (su0 + (B - 1) * su1) / nbg + se / B
    ex2 = (sq0 + 2.0 * e0 * su0 + G * e0 * e0
           + (B - 1) * sq1 + 2.0 * se1 * su1 + G * see) / nbg
    scpb = pbg[...] / jnp.sqrt(ex2 - m * m + 1e-5)
    off = pbb[...] - m * scpb
    vp_o[...] = jnp.concatenate([u0 * scpb + off, u1 * scpb + off], axis=1)
    f_o[...] = e * scpb


def _tc_pass1_body(vp, f, sum_o, m2_o):
    b = pl.program_id(0)
    v = jnp.where(b == 0, vp[:, :H], vp[:, H:])
    r = jnp.maximum(v + f[0], 0.0)

    @pl.when(b == 0)
    def _():
        sum_o[...] = jnp.zeros_like(sum_o)
        m2_o[...] = jnp.zeros_like(m2_o)

    sum_o[...] += jnp.sum(r, 0, keepdims=True)
    m2_o[...] += lax.dot_general(r, r, (((0,), (0,)), ((), ())),
                                 preferred_element_type=jnp.float32)


def _tc_pass2_body(vp, f, sum_r, m2, w1, b1, g1, bb1, w2, b2,
                   iw1, o1_o):
    nbg = float(B * G)
    mean_r = sum_r[...] / nbg
    a = m2[...] / nbg
    mr_w = _dot(mean_r, w1[...])
    mh = mr_w + b1[...]
    t = _dot(a, w1[...])
    ex2 = (jnp.sum(w1[...] * t, 0, keepdims=True)
           + 2.0 * b1[...] * mr_w + b1[...] * b1[...])
    scr = g1[...] / jnp.sqrt(ex2 - mh * mh + 1e-5)

    b = pl.program_id(0)
    v = jnp.where(b == 0, vp[:, :H], vp[:, H:])
    r = jnp.maximum(v + f[0], 0.0)
    h = _dot(r, w1[...]) + b1[...]
    h = jnp.maximum((h - mh) * scr + bb1[...], 0.0)
    oh = _dot(h, w2[...]) + b2[...]
    o1_o[...] = jnp.sum(oh * iw1[...], 1, keepdims=True)


def _tc_final_body(o1, ib1r, cgw1, cgg, cgb, cgw2, cgb2, w2a, w2bt, ib2,
                   xres, out_o):
    o1b = o1[...] + ib1r[...]
    cgp = _dot(o1b, cgw1[...])
    m = jnp.mean(cgp, 0, keepdims=True)
    v = jnp.mean(cgp * cgp, 0, keepdims=True) - m * m
    c = jnp.maximum((cgp - m) / jnp.sqrt(v + 1e-5) * cgg[...] + cgb[...], 0.0)
    cgv = _dot(c, cgw2[...]) + cgb2[...]
    out_o[...] = (o1b * w2a[...] + _dot(cgv, w2bt[...])
                  + ib2[...] + xres[...])


def _vspec(shape, imap=None):
    if imap is None:
        return pl.BlockSpec(shape, lambda b: tuple(0 for _ in shape))
    return pl.BlockSpec(shape, imap)


def _pad_edges(src, dst, ew, e_pad, n_nodes, nch):
    npd = e_pad - src.shape[0]
    fill = (jnp.arange(npd, dtype=jnp.int32) % n_nodes).astype(jnp.int32)
    src_p = jnp.concatenate([src.astype(jnp.int32), fill])
    dst_p = jnp.concatenate([dst.astype(jnp.int32), fill])
    ew_p = jnp.concatenate([ew, jnp.zeros((npd,), jnp.float32)])
    eww = e_pad // NW
    return (src_p.reshape(NW, nch, CH), dst_p.reshape(NW, nch, CH),
            ew_p.reshape(NW, eww))


def kernel(gene_expression, pert_idx, graph_batch_indices, G_coexpress,
           G_coexpress_weight, G_go, G_go_weight, params):
    del graph_batch_indices
    p = params
    f32 = jnp.float32

    src3c, dst3c, ew2c = _pad_edges(G_coexpress[0], G_coexpress[1],
                                    G_coexpress_weight, EWC * NW, G, NCHC)
    src3g, dst3g, ew2g = _pad_edges(G_go[0], G_go[1],
                                    G_go_weight, EWG * NW, P, NCHG)

    zer16c = jnp.zeros((GN, 16), f32)
    zer16g = jnp.zeros((PN, 16), f32)
    zer64c = jnp.zeros((GN, H), f32)
    zer64g = jnp.zeros((PN, H), f32)

    degc = _sc_deg_co(dst3c, ew2c, zer16c)
    degg = _sc_deg_go(dst3g, ew2g, zer16g)

    r1 = lambda a: a.reshape(1, -1)
    co_pack = jnp.concatenate(
        [p['emb_pos'], degc[0, :G, 0:1], degc[1, :G, 0:1]], axis=1)
    go_pack = jnp.concatenate(
        [p['pert_emb'], degg[0, :P, 0:1], degg[1, :P, 0:1]], axis=1)

    prep = pl.pallas_call(
        _tc_prep_body,
        out_shape=[jax.ShapeDtypeStruct((G, 2 * H), f32),
                   jax.ShapeDtypeStruct((G, H), f32),
                   jax.ShapeDtypeStruct((G, 2 * H), f32),
                   jax.ShapeDtypeStruct((P, H), f32),
                   jax.ShapeDtypeStruct((P, 2 * H), f32)],
    )
    ab, xpco, xd, xpgo, xdg = prep(
        p['gene_emb'], co_pack, go_pack,
        r1(p['bn_emb_g']), r1(p['bn_emb_b']))

    sco = _sc_agg_co(xpco, src3c, dst3c, ew2c, zer64c)
    sgo = _sc_agg_go(xpgo, src3g, dst3g, ew2g, zer64g)
    spc = jnp.concatenate([sco[0, :G], sco[1, :G]], axis=1)
    spg = jnp.concatenate([sgo[0, :P], sgo[1, :P]], axis=1)

    mida = pl.pallas_call(
        _tc_mida_body,
        out_shape=jax.ShapeDtypeStruct((G, 2 * H), f32),
    )
    up = mida(
        spc, ab, xd,
        p['sg_co_W'], r1(p['sg_co_b']),
        p['etv2_W1'], r1(p['etv2_b1']), r1(p['etv2_bng']), r1(p['etv2_bnb']),
        p['etv2_W2'], r1(p['etv2_b2']))

    midb = pl.pallas_call(
        _tc_midb_body,
        out_shape=[jax.ShapeDtypeStruct((G, 2 * H), f32),
                   jax.ShapeDtypeStruct((B, H), f32)],
    )
    vpk, f = midb(
        up, spg, xdg,
        pert_idx.reshape(2 * B, 1).astype(jnp.int32),
        p['sg_go_W'], r1(p['sg_go_b']),
        p['pf_W1'], r1(p['pf_b1']), r1(p['pf_bng']), r1(p['pf_bnb']),
        p['pf_W2'], r1(p['pf_b2']),
        r1(p['bn_pb_g']), r1(p['bn_pb_b']))

    f3 = f.reshape(B, 1, H)
    sum_r, m2 = pl.pallas_call(
        _tc_pass1_body,
        grid=(B,),
        in_specs=[_vspec((G, 2 * H)),
                  _vspec((1, 1, H), lambda b: (b, 0, 0))],
        out_specs=[_vspec((1, H)), _vspec((H, H))],
        out_shape=[jax.ShapeDtypeStruct((1, H), f32),
                   jax.ShapeDtypeStruct((H, H), f32)],
    )(vpk, f3)

    o1c = pl.pallas_call(
        _tc_pass2_body,
        grid=(B,),
        in_specs=[_vspec((G, 2 * H)),
                  _vspec((1, 1, H), lambda b: (b, 0, 0)),
                  _vspec((1, H)), _vspec((H, H)),
                  _vspec((H, 2 * H)), _vspec((1, 2 * H)),
                  _vspec((1, 2 * H)), _vspec((1, 2 * H)),
                  _vspec((2 * H, H)), _vspec((1, H)),
                  _vspec((G, H))],
        out_specs=pl.BlockSpec((G, 1), lambda b: (b, 0)),
        out_shape=jax.ShapeDtypeStruct((B * G, 1), f32),
    )(vpk, f3, sum_r, m2,
      p['rw_W1'], r1(p['rw_b1']), r1(p['rw_bng']), r1(p['rw_bnb']),
      p['rw_W2'], r1(p['rw_b2']),
      p['indv_w1'][:, :, 0])

    final = pl.pallas_call(
        _tc_final_body,
        out_shape=jax.ShapeDtypeStruct((B, G), f32),
    )
    w2 = p['indv_w2'][0]
    return final(o1c.reshape(B, G), p['indv_b1'].reshape(1, G), p['cg_W1'],
                 r1(p['cg_bng']), r1(p['cg_bnb']), p['cg_W2'], r1(p['cg_b2']),
                 w2[:, 0].reshape(1, G), w2[:, 1:].T, p['indv_b2'],
                 gene_expression.reshape(B, G))


# speedup vs baseline: 32.3896x; 1.1638x over previous
import functools
import jax
import jax.numpy as jnp
from jax import lax
from jax.experimental import pallas as pl
from jax.experimental.pallas import tpu as pltpu
from jax.experimental.pallas import tpu_sc as plsc

B = 32
G = 10000
P = 5000
H = 64
ECO = 320000
EGO = 160000
NC = 2
NS = 16
NW = NC * NS
CH = 128

EWC = 10240
EWG = 5120
NCHC = EWC // CH
NCHG = EWG // CH
GN = 10112
PN = 5120


def _make_sc_deg(npad, eww, nch):
    rp = npad // NS
    mesh = plsc.VectorSubcoreMesh(core_axis_name="c", subcore_axis_name="s")

    @functools.partial(
        pl.kernel, mesh=mesh,
        compiler_params=pltpu.CompilerParams(needs_layout_passes=False,
                                             use_tc_tiling_on_sc=False),
        out_type=jax.ShapeDtypeStruct((NC, npad, 16), jnp.float32),
        scratch_types=[
            pltpu.VMEM((nch, CH), jnp.int32),
            pltpu.VMEM((eww,), jnp.float32),
            pltpu.VMEM((CH, 16), jnp.float32),
            pltpu.VMEM_SHARED((npad, 16), jnp.float32),
        ])
    def k(dst3, ew2, zer, out, dstv, ewv, upd, acc):
        cid = lax.axis_index("c")
        sid = lax.axis_index("s")
        wid = sid * NC + cid
        pltpu.sync_copy(dst3.at[wid], dstv)
        pltpu.sync_copy(ew2.at[wid], ewv)
        pltpu.sync_copy(zer.at[pl.ds(sid * rp, rp)], acc.at[pl.ds(sid * rp, rp)])
        plsc.subcore_barrier()

        def chunk(j, carry):
            def srow(r8, c2):
                for u in range(8):
                    r = r8 * 8 + u
                    sp = plsc.load_gather(
                        ewv, [jnp.full((16,), j * CH + r, jnp.int32)])
                    upd[r, pl.ds(0, 16)] = sp
                return c2

            lax.fori_loop(0, CH // 8, srow, 0)
            pltpu.sync_copy(upd, acc.at[dstv.at[j]], add=True)
            return carry

        lax.fori_loop(0, nch, chunk, 0)
        plsc.subcore_barrier()
        pltpu.sync_copy(acc.at[pl.ds(sid * rp, rp)],
                        out.at[cid, pl.ds(sid * rp, rp)])

    return k


def _make_sc_agg(npad, eww, nch):
    rp = npad // NS
    mesh = plsc.VectorSubcoreMesh(core_axis_name="c", subcore_axis_name="s")

    @functools.partial(
        pl.kernel, mesh=mesh,
        compiler_params=pltpu.CompilerParams(needs_layout_passes=False,
                                             use_tc_tiling_on_sc=False),
        out_type=jax.ShapeDtypeStruct((NC, npad, H), jnp.float32),
        scratch_types=[
            pltpu.VMEM((nch, CH), jnp.int32),
            pltpu.VMEM((nch, CH), jnp.int32),
            pltpu.VMEM((eww,), jnp.float32),
            pltpu.VMEM((CH, H), jnp.float32),
            pltpu.VMEM((CH, H), jnp.float32),
            pltpu.VMEM_SHARED((npad, H), jnp.float32),
            pltpu.SemaphoreType.DMA,
            pltpu.SemaphoreType.DMA,
        ])
    def k(xp, src3, dst3, ew2, zer, out, srcv, dstv, ewv, rows0, rows1,
          acc, sem0, sem1):
        cid = lax.axis_index("c")
        sid = lax.axis_index("s")
        wid = sid * NC + cid
        pltpu.sync_copy(src3.at[wid], srcv)
        pltpu.sync_copy(dst3.at[wid], dstv)
        pltpu.sync_copy(ew2.at[wid], ewv)
        pltpu.sync_copy(zer.at[pl.ds(sid * rp, rp)], acc.at[pl.ds(sid * rp, rp)])
        plsc.subcore_barrier()

        def scale_scatter(j, rows):
            def srow(r4, c2):
                for u in range(4):
                    r = r4 * 4 + u
                    sp = plsc.load_gather(
                        ewv, [jnp.full((16,), j * CH + r, jnp.int32)])
                    for q in range(H // 16):
                        rows[r, pl.ds(q * 16, 16)] = (
                            rows[r, pl.ds(q * 16, 16)] * sp)
                return c2

            lax.fori_loop(0, CH // 4, srow, 0)
            pltpu.sync_copy(rows, acc.at[dstv.at[j]], add=True)

        pltpu.async_copy(xp.at[srcv.at[0]], rows0, sem0)

        def pair(i, carry):
            j0 = 2 * i
            j1 = j0 + 1
            pltpu.async_copy(xp.at[srcv.at[j1]], rows1, sem1)
            pltpu.make_async_copy(xp.at[srcv.at[j0]], rows0, sem0).wait()
            scale_scatter(j0, rows0)
            jn = jnp.where(j0 + 2 < nch, j0 + 2, 0)
            pltpu.async_copy(xp.at[srcv.at[jn]], rows0, sem0)
            pltpu.make_async_copy(xp.at[srcv.at[j1]], rows1, sem1).wait()
            scale_scatter(j1, rows1)
            return carry

        lax.fori_loop(0, nch // 2, pair, 0)
        pltpu.make_async_copy(xp.at[srcv.at[0]], rows0, sem0).wait()
        plsc.subcore_barrier()
        pltpu.sync_copy(acc.at[pl.ds(sid * rp, rp)],
                        out.at[cid, pl.ds(sid * rp, rp)])

    return k


_sc_deg_co = _make_sc_deg(GN, EWC, NCHC)
_sc_deg_go = _make_sc_deg(PN, EWG, NCHG)
_sc_agg_co = _make_sc_agg(GN, EWC, NCHC)
_sc_agg_go = _make_sc_agg(PN, EWG, NCHG)


def _normrows(x):
    n = jnp.sqrt(jnp.sum(x * x, axis=1, keepdims=True))
    return x * jnp.where(n > 1.0, 1.0 / jnp.maximum(n, 1e-7), 1.0)


def _dot(a, b):
    return jnp.dot(a, b, preferred_element_type=jnp.float32)


def _tc_prep_body(ge, cop, gop, bng, bnb,
                  ab_o, xpco_o, xd_o, xpgo_o, xdg_o):
    xn = _normrows(ge[...])
    m = jnp.mean(xn, axis=0, keepdims=True)
    v = jnp.mean(xn * xn, axis=0, keepdims=True) - m * m
    base = jnp.maximum(
        (xn - m) / jnp.sqrt(v + 1e-5) * bng[...] + bnb[...], 0.0)

    xp = _normrows(cop[:, :H])
    deg = 1.0 + cop[:, H:H + 1] + cop[:, H + 1:H + 2]
    dinv = lax.rsqrt(deg)
    ab_o[...] = jnp.concatenate([base, xp], axis=1)
    xpco_o[...] = xp * dinv
    xd_o[...] = jnp.concatenate(
        [xp * (1.0 / deg), dinv, jnp.zeros((G, H - 1), jnp.float32)], axis=1)

    xq = _normrows(gop[:, :H])
    degp = 1.0 + gop[:, H:H + 1] + gop[:, H + 1:H + 2]
    dinvp = lax.rsqrt(degp)
    xpgo_o[...] = xq * dinvp
    xdg_o[...] = jnp.concatenate(
        [xq * (1.0 / degp), dinvp, jnp.zeros((P, H - 1), jnp.float32)], axis=1)


def _tc_mida_body(spc, ab, xd,
                  wco, bco,
                  e2w1, e2b1, e2g, e2b, e2w2, e2b2,
                  up_o):
    base = ab[:, :H]
    xpos = ab[:, H:]
    agg = xd[:, H:H + 1] * (spc[:, :H] + spc[:, H:]) + xd[:, :H]
    pos0 = _dot(agg, wco[...]) + bco[...]
    pos1 = _dot(xpos, wco[...]) + bco[...]
    t0 = base + 0.2 * pos0
    t1 = base + 0.2 * pos1

    h0 = _dot(t0, e2w1[...]) + e2b1[...]
    h1 = _dot(t1, e2w1[...]) + e2b1[...]
    nbg = float(B * G)
    s1 = (jnp.sum(h0, 0, keepdims=True)
          + (B - 1) * jnp.sum(h1, 0, keepdims=True)) / nbg
    s2 = (jnp.sum(h0 * h0, 0, keepdims=True)
          + (B - 1) * jnp.sum(h1 * h1, 0, keepdims=True)) / nbg
    sc = e2g[...] / jnp.sqrt(s2 - s1 * s1 + 1e-5)
    a0 = jnp.maximum((h0 - s1) * sc + e2b[...], 0.0)
    a1 = jnp.maximum((h1 - s1) * sc + e2b[...], 0.0)
    u0 = jnp.maximum(_dot(a0, e2w2[...]) + e2b2[...], 0.0)
    u1 = jnp.maximum(_dot(a1, e2w2[...]) + e2b2[...], 0.0)
    up_o[...] = jnp.concatenate([u0, u1], axis=1)


def _tc_midb_body(up, spg, xdg, pidx,
                  wgo, bgo,
                  pfw1, pfb1, pfg, pfb, pfw2, pfb2,
                  pbg, pbb,
                  vp_o, f_o):
    pg = _dot(xdg[:, H:H + 1] * (spg[:, :H] + spg[:, H:]) + xdg[:, :H],
              wgo[...]) + bgo[...]
    iot = lax.broadcasted_iota(jnp.int32, (2 * B, P), 1)
    oh = (iot == pidx[...]).astype(jnp.float32)
    rows = _dot(oh, pg)
    ps = rows.reshape(B, 2, H).sum(axis=1)
    hp = _dot(ps, pfw1[...]) + pfb1[...]
    mp = jnp.mean(hp, 0, keepdims=True)
    vp = jnp.mean(hp * hp, 0, keepdims=True) - mp * mp
    hp = (hp - mp) / jnp.sqrt(vp + 1e-5) * pfg[...] + pfb[...]
    e = jnp.maximum(_dot(jnp.maximum(hp, 0.0), pfw2[...]) + pfb2[...], 0.0)

    u0 = up[:, :H]
    u1 = up[:, H:]
    nbg = float(B * G)
    su0 = jnp.sum(u0, 0, keepdims=True)
    su1 = jnp.sum(u1, 0, keepdims=True)
    sq0 = jnp.sum(u0 * u0, 0, keepdims=True)
    sq1 = jnp.sum(u1 * u1, 0, keepdims=True)
    se = jnp.sum(e, 0, keepdims=True)
    e0 = e[0:1]
    se1 = se - e0
    see = jnp.sum(e * e, 0, keepdims=True) - e0 * e0
    m = (su0 + (B - 1) * su1) / nbg + se / B
    ex2 = (sq0 + 2.0 * e0 * su0 + G * e0 * e0
           + (B - 1) * sq1 + 2.0 * se1 * su1 + G * see) / nbg
    scpb = pbg[...] / jnp.sqrt(ex2 - m * m + 1e-5)
    off = pbb[...] - m * scpb
    vp_o[...] = jnp.concatenate([u0 * scpb + off, u1 * scpb + off], axis=1)
    f_o[...] = e * scpb


def _tc_pass1_body(vp, f, sum_o, m2_o):
    b = pl.program_id(0)
    v = jnp.where(b == 0, vp[:, :H], vp[:, H:])
    r = jnp.maximum(v + f[0], 0.0)

    @pl.when(b == 0)
    def _():
        sum_o[...] = jnp.zeros_like(sum_o)
        m2_o[...] = jnp.zeros_like(m2_o)

    sum_o[...] += jnp.sum(r, 0, keepdims=True)
    m2_o[...] += lax.dot_general(r, r, (((0,), (0,)), ((), ())),
                                 preferred_element_type=jnp.float32)


def _tc_pass2_body(vp, f, sum_r, m2, w1, b1, g1, bb1, w2, b2,
                   iw1, o1_o):
    nbg = float(B * G)
    mean_r = sum_r[...] / nbg
    a = m2[...] / nbg
    mr_w = _dot(mean_r, w1[...])
    mh = mr_w + b1[...]
    t = _dot(a, w1[...])
    ex2 = (jnp.sum(w1[...] * t, 0, keepdims=True)
           + 2.0 * b1[...] * mr_w + b1[...] * b1[...])
    scr = g1[...] / jnp.sqrt(ex2 - mh * mh + 1e-5)

    b = pl.program_id(0)
    v = jnp.where(b == 0, vp[:, :H], vp[:, H:])
    r = jnp.maximum(v + f[0], 0.0)
    h = _dot(r, w1[...]) + b1[...]
    h = jnp.maximum((h - mh) * scr + bb1[...], 0.0)
    oh = _dot(h, w2[...]) + b2[...]
    o1_o[...] = jnp.sum(oh * iw1[...], 1, keepdims=True)


def _tc_final_body(o1, ib1r, cgw1, cgg, cgb, cgw2, cgb2, w2a, w2bt, ib2,
                   xres, out_o):
    o1b = o1[...] + ib1r[...]
    cgp = _dot(o1b, cgw1[...])
    m = jnp.mean(cgp, 0, keepdims=True)
    v = jnp.mean(cgp * cgp, 0, keepdims=True) - m * m
    c = jnp.maximum((cgp - m) / jnp.sqrt(v + 1e-5) * cgg[...] + cgb[...], 0.0)
    cgv = _dot(c, cgw2[...]) + cgb2[...]
    out_o[...] = (o1b * w2a[...] + _dot(cgv, w2bt[...])
                  + ib2[...] + xres[...])


def _vspec(shape, imap=None):
    if imap is None:
        return pl.BlockSpec(shape, lambda b: tuple(0 for _ in shape))
    return pl.BlockSpec(shape, imap)


def _pad_edges(src, dst, ew, e_pad, n_nodes, nch):
    npd = e_pad - src.shape[0]
    fill = (jnp.arange(npd, dtype=jnp.int32) % n_nodes).astype(jnp.int32)
    src_p = jnp.concatenate([src.astype(jnp.int32), fill])
    dst_p = jnp.concatenate([dst.astype(jnp.int32), fill])
    ew_p = jnp.concatenate([ew, jnp.zeros((npd,), jnp.float32)])
    eww = e_pad // NW
    return (src_p.reshape(NW, nch, CH), dst_p.reshape(NW, nch, CH),
            ew_p.reshape(NW, eww))


def kernel(gene_expression, pert_idx, graph_batch_indices, G_coexpress,
           G_coexpress_weight, G_go, G_go_weight, params):
    del graph_batch_indices
    p = params
    f32 = jnp.float32

    src3c, dst3c, ew2c = _pad_edges(G_coexpress[0], G_coexpress[1],
                                    G_coexpress_weight, EWC * NW, G, NCHC)
    src3g, dst3g, ew2g = _pad_edges(G_go[0], G_go[1],
                                    G_go_weight, EWG * NW, P, NCHG)

    zer16c = jnp.zeros((GN, 16), f32)
    zer16g = jnp.zeros((PN, 16), f32)
    zer64c = jnp.zeros((GN, H), f32)
    zer64g = jnp.zeros((PN, H), f32)

    degc = _sc_deg_co(dst3c, ew2c, zer16c)
    degg = _sc_deg_go(dst3g, ew2g, zer16g)

    r1 = lambda a: a.reshape(1, -1)
    co_pack = jnp.concatenate(
        [p['emb_pos'], degc[0, :G, 0:1], degc[1, :G, 0:1]], axis=1)
    go_pack = jnp.concatenate(
        [p['pert_emb'], degg[0, :P, 0:1], degg[1, :P, 0:1]], axis=1)

    prep = pl.pallas_call(
        _tc_prep_body,
        out_shape=[jax.ShapeDtypeStruct((G, 2 * H), f32),
                   jax.ShapeDtypeStruct((G, H), f32),
                   jax.ShapeDtypeStruct((G, 2 * H), f32),
                   jax.ShapeDtypeStruct((P, H), f32),
                   jax.ShapeDtypeStruct((P, 2 * H), f32)],
    )
    ab, xpco, xd, xpgo, xdg = prep(
        p['gene_emb'], co_pack, go_pack,
        r1(p['bn_emb_g']), r1(p['bn_emb_b']))

    sco = _sc_agg_co(xpco, src3c, dst3c, ew2c, zer64c)
    sgo = _sc_agg_go(xpgo, src3g, dst3g, ew2g, zer64g)
    spc = jnp.concatenate([sco[0, :G], sco[1, :G]], axis=1)
    spg = jnp.concatenate([sgo[0, :P], sgo[1, :P]], axis=1)

    mida = pl.pallas_call(
        _tc_mida_body,
        out_shape=jax.ShapeDtypeStruct((G, 2 * H), f32),
    )
    up = mida(
        spc, ab, xd,
        p['sg_co_W'], r1(p['sg_co_b']),
        p['etv2_W1'], r1(p['etv2_b1']), r1(p['etv2_bng']), r1(p['etv2_bnb']),
        p['etv2_W2'], r1(p['etv2_b2']))

    midb = pl.pallas_call(
        _tc_midb_body,
        out_shape=[jax.ShapeDtypeStruct((G, 2 * H), f32),
                   jax.ShapeDtypeStruct((B, H), f32)],
    )
    vpk, f = midb(
        up, spg, xdg,
        pert_idx.reshape(2 * B, 1).astype(jnp.int32),
        p['sg_go_W'], r1(p['sg_go_b']),
        p['pf_W1'], r1(p['pf_b1']), r1(p['pf_bng']), r1(p['pf_bnb']),
        p['pf_W2'], r1(p['pf_b2']),
        r1(p['bn_pb_g']), r1(p['bn_pb_b']))

    f3 = f.reshape(B, 1, H)
    sum_r, m2 = pl.pallas_call(
        _tc_pass1_body,
        grid=(B,),
        in_specs=[_vspec((G, 2 * H)),
                  _vspec((1, 1, H), lambda b: (b, 0, 0))],
        out_specs=[_vspec((1, H)), _vspec((H, H))],
        out_shape=[jax.ShapeDtypeStruct((1, H), f32),
                   jax.ShapeDtypeStruct((H, H), f32)],
    )(vpk, f3)

    o1c = pl.pallas_call(
        _tc_pass2_body,
        grid=(B,),
        in_specs=[_vspec((G, 2 * H)),
                  _vspec((1, 1, H), lambda b: (b, 0, 0)),
                  _vspec((1, H)), _vspec((H, H)),
                  _vspec((H, 2 * H)), _vspec((1, 2 * H)),
                  _vspec((1, 2 * H)), _vspec((1, 2 * H)),
                  _vspec((2 * H, H)), _vspec((1, H)),
                  _vspec((G, H))],
        out_specs=pl.BlockSpec((G, 1), lambda b: (b, 0)),
        out_shape=jax.ShapeDtypeStruct((B * G, 1), f32),
    )(vpk, f3, sum_r, m2,
      p['rw_W1'], r1(p['rw_b1']), r1(p['rw_bng']), r1(p['rw_bnb']),
      p['rw_W2'], r1(p['rw_b2']),
      p['indv_w1'][:, :, 0])

    final = pl.pallas_call(
        _tc_final_body,
        out_shape=jax.ShapeDtypeStruct((B, G), f32),
    )
    w2 = p['indv_w2'][0]
    return final(o1c.reshape(B, G), p['indv_b1'].reshape(1, G), p['cg_W1'],
                 r1(p['cg_bng']), r1(p['cg_bnb']), p['cg_W2'], r1(p['cg_b2']),
                 w2[:, 0].reshape(1, G), w2[:, 1:].T, p['indv_b2'],
                 gene_expression.reshape(B, G))
